# baseline (speedup 1.0000x reference)
import functools

import jax
import jax.numpy as jnp
from jax import lax
from jax.experimental import pallas as pl
from jax.experimental.pallas import tpu as pltpu
from jax.experimental.pallas import tpu_sc as plsc

BATCH = 16384
EMBED_DIM = 64
LANES = 16


def _mf_body(uid_hbm, cid_hbm, uemb_hbm, cemb_hbm, ub_hbm, cb_hbm, gb_hbm,
             out_hbm,
             uidx_v, cidx_v, urows_v, crows_v, ubias_v, cbias_v, gb_v, out_v,
             sem_u, sem_c, sem_ub, sem_cb):
    nc = plsc.get_sparse_core_info().num_cores
    wid = lax.axis_index("s") * nc + lax.axis_index("c")
    bpw = BATCH // (nc * plsc.get_sparse_core_info().num_subcores)
    base = wid * bpw

    pltpu.sync_copy(uid_hbm.at[pl.ds(base, bpw)], uidx_v)
    pltpu.sync_copy(cid_hbm.at[pl.ds(base, bpw)], cidx_v)
    cp_u = pltpu.async_copy(uemb_hbm.at[uidx_v], urows_v, sem_u)
    cp_c = pltpu.async_copy(cemb_hbm.at[cidx_v], crows_v, sem_c)
    cp_ub = pltpu.async_copy(ub_hbm.at[uidx_v], ubias_v, sem_ub)
    cp_cb = pltpu.async_copy(cb_hbm.at[cidx_v], cbias_v, sem_cb)
    pltpu.sync_copy(gb_hbm, gb_v)
    cp_u.wait()
    cp_c.wait()
    cp_ub.wait()
    cp_cb.wait()

    gb = gb_v[...]
    lane = lax.iota(jnp.int32, 16)
    bfly = [(lane ^ sh)[:, None] for sh in (8, 4, 2, 1)]
    dnums = lax.GatherDimensionNumbers(
        offset_dims=(), collapsed_slice_dims=(0,), start_index_map=(0,))

    def hsum(p):
        for idx in bfly:
            p = p + lax.gather(p, idx, dnums, (1,),
                               mode=lax.GatherScatterMode.PROMISE_IN_BOUNDS)
        return p

    def j_body(j, carry):
        rbase = j * LANES
        acc = jnp.zeros((LANES,), jnp.float32)
        for i in range(LANES):
            r = rbase + i
            p = urows_v[r, pl.ds(0, 16)] * crows_v[r, pl.ds(0, 16)]
            for k in range(1, EMBED_DIM // 16):
                p = p + urows_v[r, pl.ds(16 * k, 16)] * crows_v[r, pl.ds(16 * k, 16)]
            acc = jnp.where(lane == i, hsum(p), acc)
        out_v[pl.ds(rbase, LANES)] = (
            acc + ubias_v[pl.ds(rbase, LANES)] + cbias_v[pl.ds(rbase, LANES)] + gb
        )
        return carry

    lax.fori_loop(0, bpw // LANES, j_body, 0)

    pltpu.sync_copy(out_v, out_hbm.at[pl.ds(base, bpw)])


def kernel(user_ids, course_ids, user_embedding, course_embedding,
           user_bias, course_bias, global_bias):
    info = plsc.get_sparse_core_info()
    nw = info.num_cores * info.num_subcores
    bpw = BATCH // nw
    mesh = plsc.VectorSubcoreMesh(core_axis_name="c", subcore_axis_name="s")

    uids = user_ids.astype(jnp.int32)
    cids = course_ids.astype(jnp.int32)
    ub = user_bias.reshape(-1)
    cb = course_bias.reshape(-1)
    gb = jnp.broadcast_to(global_bias, (LANES,))

    run = pl.kernel(
        _mf_body,
        mesh=mesh,
        compiler_params=pltpu.CompilerParams(use_tc_tiling_on_sc=False),
        out_type=jax.ShapeDtypeStruct((BATCH,), jnp.float32),
        scratch_types=[
            pltpu.VMEM((bpw,), jnp.int32),
            pltpu.VMEM((bpw,), jnp.int32),
            pltpu.VMEM((bpw, EMBED_DIM), jnp.float32),
            pltpu.VMEM((bpw, EMBED_DIM), jnp.float32),
            pltpu.VMEM((bpw,), jnp.float32),
            pltpu.VMEM((bpw,), jnp.float32),
            pltpu.VMEM((LANES,), jnp.float32),
            pltpu.VMEM((bpw,), jnp.float32),
            pltpu.SemaphoreType.DMA,
            pltpu.SemaphoreType.DMA,
            pltpu.SemaphoreType.DMA,
            pltpu.SemaphoreType.DMA,
        ],
    )
    return run(uids, cids, user_embedding, course_embedding, ub, cb, gb)


# speedup vs baseline: 1.4480x; 1.4480x over previous
import functools

import jax
import jax.numpy as jnp
from jax import lax
from jax.experimental import pallas as pl
from jax.experimental.pallas import tpu as pltpu
from jax.experimental.pallas import tpu_sc as plsc

BATCH = 16384
EMBED_DIM = 64
LANES = 16
CHUNK = 64


def _mf_body(uid_hbm, cid_hbm, uemb_hbm, cemb_hbm, ub_hbm, cb_hbm, gb_hbm,
             out_hbm,
             uidx_v, cidx_v, ubuf_v, cbuf_v, ubias_v, cbias_v, gb_v, out_v,
             sem_u, sem_c, sem_ub, sem_cb):
    info = plsc.get_sparse_core_info()
    nc = info.num_cores
    bpw = BATCH // (nc * info.num_subcores)
    wid = lax.axis_index("s") * nc + lax.axis_index("c")
    base = wid * bpw
    nvec = bpw // LANES

    pltpu.sync_copy(uid_hbm.at[pl.ds(base, bpw)], uidx_v)
    pltpu.sync_copy(cid_hbm.at[pl.ds(base, bpw)], cidx_v)
    cp_ub = pltpu.async_copy(ub_hbm.at[uidx_v], ubias_v, sem_ub)
    cp_cb = pltpu.async_copy(cb_hbm.at[cidx_v], cbias_v, sem_cb)
    pltpu.sync_copy(gb_hbm, gb_v)

    cp_ub.wait()
    cp_cb.wait()

    gb = gb_v[...]
    lane = lax.iota(jnp.int32, 16)
    bfly = [(lane ^ sh)[:, None] for sh in (8, 4, 2, 1)]
    dnums = lax.GatherDimensionNumbers(
        offset_dims=(), collapsed_slice_dims=(0,), start_index_map=(0,))

    def hsum(p):
        for idx in bfly:
            p = p + lax.gather(p, idx, dnums, (1,),
                               mode=lax.GatherScatterMode.PROMISE_IN_BOUNDS)
        return p

    def chunk_body(ch, carry):
        vb = ch * CHUNK
        for jv in range(CHUNK // LANES):
            vu = uidx_v[pl.ds(vb + jv * LANES, LANES)]
            vc = cidx_v[pl.ds(vb + jv * LANES, LANES)]
            for i in range(LANES):
                slot = jv * LANES + i
                pltpu.async_copy(uemb_hbm.at[vu[i]], ubuf_v.at[slot], sem_u)
                pltpu.async_copy(cemb_hbm.at[vc[i]], cbuf_v.at[slot], sem_c)
        for jv in range(CHUNK // LANES):
            for i in range(LANES):
                slot = jv * LANES + i
                pltpu.make_async_copy(
                    uemb_hbm.at[0], ubuf_v.at[slot], sem_u).wait()
                pltpu.make_async_copy(
                    cemb_hbm.at[0], cbuf_v.at[slot], sem_c).wait()
        for jv in range(CHUNK // LANES):
            acc = jnp.zeros((LANES,), jnp.float32)
            for i in range(LANES):
                slot = jv * LANES + i
                p = ubuf_v[slot, pl.ds(0, 16)] * cbuf_v[slot, pl.ds(0, 16)]
                for k in range(1, EMBED_DIM // 16):
                    p = p + (ubuf_v[slot, pl.ds(16 * k, 16)]
                             * cbuf_v[slot, pl.ds(16 * k, 16)])
                acc = jnp.where(lane == i, hsum(p), acc)
            rbase = vb + jv * LANES
            out_v[pl.ds(rbase, LANES)] = (
                acc + ubias_v[pl.ds(rbase, LANES)]
                + cbias_v[pl.ds(rbase, LANES)] + gb
            )
        return carry

    lax.fori_loop(0, bpw // CHUNK, chunk_body, 0)

    pltpu.sync_copy(out_v, out_hbm.at[pl.ds(base, bpw)])


def kernel(user_ids, course_ids, user_embedding, course_embedding,
           user_bias, course_bias, global_bias):
    info = plsc.get_sparse_core_info()
    nw = info.num_cores * info.num_subcores
    bpw = BATCH // nw
    mesh = plsc.VectorSubcoreMesh(core_axis_name="c", subcore_axis_name="s")

    uids = user_ids.astype(jnp.int32)
    cids = course_ids.astype(jnp.int32)
    ub = user_bias.reshape(-1)
    cb = course_bias.reshape(-1)
    gb = jnp.broadcast_to(global_bias, (LANES,))

    run = pl.kernel(
        _mf_body,
        mesh=mesh,
        compiler_params=pltpu.CompilerParams(use_tc_tiling_on_sc=True),
        out_type=jax.ShapeDtypeStruct((BATCH,), jnp.float32),
        scratch_types=[
            pltpu.VMEM((bpw,), jnp.int32),
            pltpu.VMEM((bpw,), jnp.int32),
            pltpu.VMEM((CHUNK, EMBED_DIM), jnp.float32),
            pltpu.VMEM((CHUNK, EMBED_DIM), jnp.float32),
            pltpu.VMEM((bpw,), jnp.float32),
            pltpu.VMEM((bpw,), jnp.float32),
            pltpu.VMEM((LANES,), jnp.float32),
            pltpu.VMEM((bpw,), jnp.float32),
            pltpu.SemaphoreType.DMA,
            pltpu.SemaphoreType.DMA,
            pltpu.SemaphoreType.DMA,
            pltpu.SemaphoreType.DMA,
        ],
    )
    return run(uids, cids, user_embedding, course_embedding, ub, cb, gb)
